# baseline (speedup 1.0000x reference)
import functools

import jax
import jax.numpy as jnp
from jax import lax
from jax.experimental import pallas as pl
from jax.experimental.pallas import tpu as pltpu
from jax.experimental.pallas import tpu_sc as plsc

D = 64
BLK = 128
S = 50
NB1 = 16384
CPW = 4


@functools.lru_cache(maxsize=None)
def _build():
    info = plsc.get_sparse_core_info()
    NC = info.num_cores
    NW = NC * info.num_subcores
    n_blocks = S * CPW
    mesh = plsc.VectorSubcoreMesh(core_axis_name="c", subcore_axis_name="s")

    @functools.partial(
        pl.kernel,
        mesh=mesh,
        out_type=jax.ShapeDtypeStruct((S, D, NB1), jnp.float32),
        compiler_params=pltpu.CompilerParams(use_tc_tiling_on_sc=True,
                                             needs_layout_passes=False),
        scratch_types=[
            pltpu.VMEM((S, CPW * BLK), jnp.int32),
            pltpu.VMEM((2, BLK), jnp.int32),
            pltpu.VMEM((2, BLK, BLK), jnp.float32),
            pltpu.VMEM((2, D, BLK), jnp.float32),
            pltpu.SemaphoreType.DMA,
            pltpu.SemaphoreType.DMA,
            pltpu.SemaphoreType.DMA,
            pltpu.SemaphoreType.DMA,
        ],
    )
    def gather_kernel(ids_hbm, tab_hbm, out_hbm, slab, pairs, block, stag,
                      sg0, sg1, sw0, sw1):
        sem_g = (sg0, sg1)
        sem_w = (sw0, sw1)
        wid = lax.axis_index("s") * NC + lax.axis_index("c")
        col0 = wid * (CPW * BLK)

        pltpu.sync_copy(ids_hbm.at[:, pl.ds(col0, CPW * BLK)], slab)

        iotas = [lax.iota(jnp.int32, 16) + 16 * g for g in range(8)]

        def compute_pairs(k, par):
            s = k >> 2
            off = (k & 3) * BLK
            for g in range(8):
                v = slab[s, pl.ds(off + 16 * g, 16)]
                pairs[par, pl.ds(16 * g, 16)] = v >> 1

        def issue_gather(par):
            pltpu.async_copy(tab_hbm.at[pairs.at[par]], block.at[par],
                             sem_g[par])

        def wait_gather(par):
            pltpu.make_async_copy(tab_hbm.at[pairs.at[par]], block.at[par],
                                  sem_g[par]).wait()

        def issue_write(k, par):
            s = k >> 2
            c = (k & 3) * BLK + col0
            pltpu.async_copy(stag.at[par], out_hbm.at[s, :, pl.ds(c, BLK)],
                             sem_w[par])

        def wait_write(k, par):
            s = k >> 2
            c = (k & 3) * BLK + col0
            pltpu.make_async_copy(stag.at[par],
                                  out_hbm.at[s, :, pl.ds(c, BLK)],
                                  sem_w[par]).wait()

        def transpose_block(k, par):
            s = k >> 2
            off = (k & 3) * BLK
            hcs = tuple((slab[s, pl.ds(off + 16 * g, 16)] & 1) * D
                        for g in range(8))

            def dbody(d, carry):
                for g in range(8):
                    val = plsc.load_gather(block.at[par],
                                           [iotas[g], hcs[g] + d])
                    stag[par, d, pl.ds(16 * g, 16)] = val
                return carry

            lax.fori_loop(0, D, dbody, 0)

        for par in range(2):
            compute_pairs(par, par)
            issue_gather(par)

        def body(kk, carry):
            for par in range(2):
                k = 2 * kk + par
                wait_gather(par)

                @pl.when(k >= 2)
                def _():
                    wait_write(k - 2, par)

                transpose_block(k, par)
                issue_write(k, par)

                @pl.when(k < n_blocks - 2)
                def _():
                    compute_pairs(k + 2, par)
                    issue_gather(par)
            return carry

        lax.fori_loop(0, n_blocks // 2, body, 0)

        for par in range(2):
            wait_write(n_blocks - 2 + par, par)

    return gather_kernel


def kernel(token_ids, embedding_matrix):
    ids_t = token_ids.T.astype(jnp.int32)
    tab2 = embedding_matrix.reshape(500000, 128)
    out3 = _build()(ids_t, tab2)
    return jnp.transpose(out3, (2, 0, 1))


# speedup vs baseline: 1.1460x; 1.1460x over previous
import functools

import jax
import jax.numpy as jnp
from jax import lax
from jax.experimental import pallas as pl
from jax.experimental.pallas import tpu as pltpu
from jax.experimental.pallas import tpu_sc as plsc

D = 64
BLK = 128
S = 50
NB1 = 16384
CPW = 4


@functools.lru_cache(maxsize=None)
def _build():
    info = plsc.get_sparse_core_info()
    NC = info.num_cores
    NW = NC * info.num_subcores
    n_blocks = S * CPW
    mesh = plsc.VectorSubcoreMesh(core_axis_name="c", subcore_axis_name="s")

    @functools.partial(
        pl.kernel,
        mesh=mesh,
        out_type=jax.ShapeDtypeStruct((S, D, NB1), jnp.float32),
        compiler_params=pltpu.CompilerParams(use_tc_tiling_on_sc=True,
                                             needs_layout_passes=False),
        scratch_types=[
            pltpu.VMEM((S, CPW * BLK), jnp.int32),
            pltpu.VMEM((2, BLK), jnp.int32),
            pltpu.VMEM((2, BLK, BLK), jnp.float32),
            pltpu.VMEM((2, D, BLK), jnp.float32),
            pltpu.SemaphoreType.DMA,
            pltpu.SemaphoreType.DMA,
            pltpu.SemaphoreType.DMA,
            pltpu.SemaphoreType.DMA,
        ],
    )
    def gather_kernel(ids_hbm, tab_hbm, out_hbm, slab, pairs, block, stag,
                      sg0, sg1, sw0, sw1):
        sem_g = (sg0, sg1)
        sem_w = (sw0, sw1)
        wid = lax.axis_index("s") * NC + lax.axis_index("c")
        col0 = wid * (CPW * BLK)

        pltpu.sync_copy(ids_hbm.at[:, pl.ds(col0, CPW * BLK)], slab)

        iotas = [lax.iota(jnp.int32, 16) + 16 * g for g in range(8)]

        def compute_pairs(k, par):
            s = k >> 2
            off = (k & 3) * BLK
            for g in range(8):
                v = slab[s, pl.ds(off + 16 * g, 16)]
                pairs[par, pl.ds(16 * g, 16)] = v >> 1

        def issue_gather(par):
            pltpu.async_copy(tab_hbm.at[pairs.at[par]], block.at[par],
                             sem_g[par])

        def wait_gather(par):
            pltpu.make_async_copy(tab_hbm.at[pairs.at[par]], block.at[par],
                                  sem_g[par]).wait()

        def issue_write(k, par):
            s = k >> 2
            c = (k & 3) * BLK + col0
            pltpu.async_copy(stag.at[par], out_hbm.at[s, :, pl.ds(c, BLK)],
                             sem_w[par])

        def wait_write(k, par):
            s = k >> 2
            c = (k & 3) * BLK + col0
            pltpu.make_async_copy(stag.at[par],
                                  out_hbm.at[s, :, pl.ds(c, BLK)],
                                  sem_w[par]).wait()

        def transpose_block(k, par):
            s = k >> 2
            off = (k & 3) * BLK
            hcs = tuple((slab[s, pl.ds(off + 16 * g, 16)] & 1) * D
                        for g in range(8))

            def dbody(i, carry):
                for dd in range(8):
                    d = i * 8 + dd
                    vals = [plsc.load_gather(block.at[par],
                                             [iotas[g], hcs[g] + d])
                            for g in range(8)]
                    for g in range(8):
                        stag[par, d, pl.ds(16 * g, 16)] = vals[g]
                return carry

            lax.fori_loop(0, D // 8, dbody, 0)

        for par in range(2):
            compute_pairs(par, par)
            issue_gather(par)

        def body(kk, carry):
            for par in range(2):
                k = 2 * kk + par
                wait_gather(par)

                @pl.when(k >= 2)
                def _():
                    wait_write(k - 2, par)

                transpose_block(k, par)
                issue_write(k, par)

                @pl.when(k < n_blocks - 2)
                def _():
                    compute_pairs(k + 2, par)
                    issue_gather(par)
            return carry

        lax.fori_loop(0, n_blocks // 2, body, 0)

        for par in range(2):
            wait_write(n_blocks - 2 + par, par)

    return gather_kernel


def kernel(token_ids, embedding_matrix):
    ids_t = token_ids.T.astype(jnp.int32)
    tab2 = embedding_matrix.reshape(500000, 128)
    out3 = _build()(ids_t, tab2)
    return jnp.transpose(out3, (2, 0, 1))
